# baseline (speedup 1.0000x reference)
import functools
import math

import jax
import jax.numpy as jnp
from jax import lax
from jax.experimental import pallas as pl
from jax.experimental.pallas import tpu as pltpu
from jax.experimental.pallas import tpu_sc as plsc

MUL = 128
HID = 64
NATTR = 10
NPASS = 5
GW = 2 * HID
AVG_NUM_NEIGHBORS = 32.0
_L_OF_D = (0, 1, 1, 1, 2, 2, 2, 2, 2)


def _mlp_body(ed_ref, w0_ref, w1_ref, w2_ref, h_ref):
    h = jax.nn.silu(jnp.dot(ed_ref[...], w0_ref[...],
                            preferred_element_type=jnp.float32) * (1.0 / math.sqrt(8.0)))
    h = jax.nn.silu(jnp.dot(h, w1_ref[...],
                            preferred_element_type=jnp.float32) * (1.0 / math.sqrt(HID)))
    h_ref[...] = jax.nn.silu(jnp.dot(h, w2_ref[...],
                                     preferred_element_type=jnp.float32) * (1.0 / math.sqrt(HID)))


def _build_sc_scatter(E, N):
    n_tiles = 32
    ept = E // n_tiles
    C = 40
    n_chunks = ept // C
    NBUF = 5
    n_outer = n_chunks // NBUF
    rows = (N + 127) // 128 * 128 + 128
    rows_per_tile = rows // 16
    mesh = plsc.VectorSubcoreMesh(core_axis_name="c", subcore_axis_name="s")
    out_t = tuple(jax.ShapeDtypeStruct((2, rows, GW), jnp.float32) for _ in range(NPASS))

    n_outer2 = n_outer // 2

    @functools.partial(
        pl.kernel, out_type=out_t, mesh=mesh,
        compiler_params=pltpu.CompilerParams(needs_layout_passes=False),
        scratch_types=[pltpu.VMEM((C, GW), jnp.float32)] * 2
                      + [pltpu.VMEM((C, HID), jnp.float32)] * NBUF
                      + [pltpu.VMEM((C,), jnp.int32)] * (2 * NBUF)
                      + [pltpu.VMEM((C,), jnp.float32)] * (2 * NBUF)
                      + [pltpu.VMEM_SHARED((rows, GW), jnp.float32)]
                      + [pltpu.SemaphoreType.DMA] * (NBUF + 2))
    def sc_scatter(src_hbm, h_hbm, sh_hbm, zero_hbm,
                   o0, o1, o2, o3, o4, *rest):
        pays = rest[:2]
        hbufs = rest[2:2 + NBUF]
        idxs = [rest[2 + NBUF:2 + 2 * NBUF], rest[2 + 2 * NBUF:2 + 3 * NBUF]]
        sh0s = rest[2 + 3 * NBUF:2 + 4 * NBUF]
        sh1s = rest[2 + 4 * NBUF:2 + 5 * NBUF]
        accum = rest[2 + 5 * NBUF]
        gsems = rest[3 + 5 * NBUF:3 + 6 * NBUF]
        ssems = rest[3 + 6 * NBUF:]
        c = lax.axis_index("c")
        s = lax.axis_index("s")
        wid = c * 16 + s
        tile_base = wid * ept
        row0 = s * rows_per_tile

        for p, o_hbm in enumerate((o0, o1, o2, o3, o4)):
            pltpu.sync_copy(zero_hbm.at[pl.ds(row0, rows_per_tile)],
                            accum.at[pl.ds(row0, rows_per_tile)])
            plsc.subcore_barrier()

            def fetch(i, q, b):
                pltpu.async_copy(src_hbm.at[wid, i], idxs[q][b], gsems[b])
                pltpu.async_copy(h_hbm.at[pl.ds(tile_base + i * C, C)],
                                 hbufs[b], gsems[b])
                pltpu.async_copy(sh_hbm.at[2 * p, wid, i], sh0s[b], gsems[b])
                pltpu.async_copy(sh_hbm.at[2 * p + 1, wid, i], sh1s[b], gsems[b])

            for b in range(NBUF):
                fetch(b, 0, b)

            def compute(b, pb):
                def edge(e, carry):
                    e16 = jnp.full((16,), e, dtype=jnp.int32)
                    b0 = plsc.load_gather(sh0s[b], [e16])
                    b1 = plsc.load_gather(sh1s[b], [e16])
                    for j in range(HID // 16):
                        hv = hbufs[b][e, pl.ds(16 * j, 16)]
                        pays[pb][e, pl.ds(16 * j, 16)] = hv * b0
                        pays[pb][e, pl.ds(HID + 16 * j, 16)] = hv * b1
                    return carry
                lax.fori_loop(0, C, edge, 0)

            def outer(j2, carry):
                for q in range(2):
                    for b in range(NBUF):
                        i = j2 * 2 * NBUF + q * NBUF + b
                        pb = (q * NBUF + b) % 2
                        pltpu.make_async_copy(src_hbm.at[0, 0], idxs[q][b],
                                              gsems[b]).wait()
                        pltpu.make_async_copy(h_hbm.at[pl.ds(0, C)], hbufs[b],
                                              gsems[b]).wait()
                        pltpu.make_async_copy(sh_hbm.at[0, 0, 0], sh0s[b],
                                              gsems[b]).wait()
                        pltpu.make_async_copy(sh_hbm.at[0, 0, 0], sh1s[b],
                                              gsems[b]).wait()
                        if q == 0 and b < 2:
                            @pl.when(j2 > 0)
                            def _():
                                pltpu.make_async_copy(
                                    pays[pb], accum.at[idxs[q][b]],
                                    ssems[pb]).wait()
                        else:
                            pltpu.make_async_copy(
                                pays[pb], accum.at[idxs[q][b]], ssems[pb]).wait()
                        compute(b, pb)
                        pltpu.async_copy(pays[pb], accum.at[idxs[q][b]],
                                         ssems[pb], add=True)
                        if q == 0:
                            fetch(i + NBUF, 1, b)
                        else:
                            @pl.when(j2 < n_outer2 - 1)
                            def _():
                                fetch(i + NBUF, 0, b)
                return carry

            lax.fori_loop(0, n_outer2, outer, 0)
            for pb in range(2):
                pltpu.make_async_copy(pays[pb], accum.at[idxs[1][0]],
                                      ssems[pb]).wait()
            plsc.subcore_barrier()
            pltpu.sync_copy(accum.at[pl.ds(row0, rows_per_tile)],
                            o_hbm.at[c, pl.ds(row0, rows_per_tile)])

    return sc_scatter


def _final_body(nf_ref, na_ref, *rest):
    t_refs = rest[:2 * NPASS]
    (wskip_ref, wlin_ref, wmlp3_ref, w20_ref, w21_ref, w22_ref,
     out9_ref, sc_ref) = rest[2 * NPASS:]
    nf = nf_ref[...]
    acc = jnp.zeros_like(nf)
    for v in range(NATTR):
        acc = acc + jnp.dot(nf, wskip_ref[:, v, :],
                            preferred_element_type=jnp.float32) * na_ref[:, v:v + 1]
    sc_ref[...] = acc * (1.0 / math.sqrt(MUL * NATTR))
    x = jnp.dot(nf, wlin_ref[...],
                preferred_element_type=jnp.float32) * (1.0 / math.sqrt(MUL))
    w2s = (w20_ref, w21_ref, w22_ref)
    scale = 1.0 / (math.sqrt(HID) * math.sqrt(MUL) * AVG_NUM_NEIGHBORS)
    for d in range(9):
        p, dl = divmod(d, 2)
        l = _L_OF_D[d]
        ta, tb = t_refs[2 * p], t_refs[2 * p + 1]
        td = (ta[0, :, HID * dl:HID * (dl + 1)]
              + tb[0, :, HID * dl:HID * (dl + 1)])
        m = jnp.dot(td, wmlp3_ref[:, l * MUL:(l + 1) * MUL],
                    preferred_element_type=jnp.float32)
        out9_ref[d] = jnp.dot(x * m, w2s[l][...],
                              preferred_element_type=jnp.float32) * scale


def kernel(node_feat, node_attr, edge_idx, edge_dist_embedding, edge_diff_embedding,
           W_skip, W_lin1, W_mlp0, W_mlp1, W_mlp2, W_mlp3, W2_0, W2_1, W2_2):
    N = node_feat.shape[0]
    E = edge_dist_embedding.shape[0]
    src = edge_idx[:, 0]
    rows = (N + 127) // 128 * 128 + 128

    EB = 8000
    h = pl.pallas_call(
        _mlp_body,
        grid=(E // EB,),
        in_specs=[
            pl.BlockSpec((EB, 8), lambda i: (i, 0)),
            pl.BlockSpec((8, HID), lambda i: (0, 0)),
            pl.BlockSpec((HID, HID), lambda i: (0, 0)),
            pl.BlockSpec((HID, HID), lambda i: (0, 0)),
        ],
        out_specs=pl.BlockSpec((EB, HID), lambda i: (i, 0)),
        out_shape=jax.ShapeDtypeStruct((E, HID), jnp.float32),
    )(edge_dist_embedding, W_mlp0, W_mlp1, W_mlp2)

    zeros = jnp.zeros((rows, GW), jnp.float32)
    shp = jnp.pad(edge_diff_embedding, ((0, 0), (0, 1)))
    sh10 = shp.T.reshape(10, 32, -1, 40)
    ts = _build_sc_scatter(E, N)(src.reshape(32, -1, 40), h, sh10, zeros)

    NB = 400
    t_specs = []
    t_args = []
    for t in ts:
        t_specs += [pl.BlockSpec((1, NB, GW), lambda i: (0, i, 0)),
                    pl.BlockSpec((1, NB, GW), lambda i: (1, i, 0))]
        t_args += [t, t]
    out9, sc = pl.pallas_call(
        _final_body,
        grid=(N // NB,),
        in_specs=[
            pl.BlockSpec((NB, MUL), lambda i: (i, 0)),
            pl.BlockSpec((NB, NATTR), lambda i: (i, 0)),
        ] + t_specs + [
            pl.BlockSpec((MUL, NATTR, MUL), lambda i: (0, 0, 0)),
            pl.BlockSpec((MUL, MUL), lambda i: (0, 0)),
            pl.BlockSpec((HID, 3 * MUL), lambda i: (0, 0)),
            pl.BlockSpec((MUL, MUL), lambda i: (0, 0)),
            pl.BlockSpec((MUL, MUL), lambda i: (0, 0)),
            pl.BlockSpec((MUL, MUL), lambda i: (0, 0)),
        ],
        out_specs=[
            pl.BlockSpec((9, NB, MUL), lambda i: (0, i, 0)),
            pl.BlockSpec((NB, MUL), lambda i: (i, 0)),
        ],
        out_shape=[
            jax.ShapeDtypeStruct((9, N, MUL), jnp.float32),
            jax.ShapeDtypeStruct((N, MUL), jnp.float32),
        ],
    )(node_feat, node_attr, *t_args,
      W_skip, W_lin1, W_mlp3, W2_0, W2_1, W2_2)

    return (jnp.transpose(out9, (1, 2, 0)), sc)


# speedup vs baseline: 1.8667x; 1.8667x over previous
import functools
import math

import jax
import jax.numpy as jnp
from jax import lax
from jax.experimental import pallas as pl
from jax.experimental.pallas import tpu as pltpu
from jax.experimental.pallas import tpu_sc as plsc

MUL = 128
HID = 64
NATTR = 10
NPASS = 5
GW = 2 * HID
AVG_NUM_NEIGHBORS = 32.0
_L_OF_D = (0, 1, 1, 1, 2, 2, 2, 2, 2)


def _mlp_body(ed_ref, w0_ref, w1_ref, w2_ref, h_ref):
    h = jax.nn.silu(jnp.dot(ed_ref[...], w0_ref[...],
                            preferred_element_type=jnp.float32) * (1.0 / math.sqrt(8.0)))
    h = jax.nn.silu(jnp.dot(h, w1_ref[...],
                            preferred_element_type=jnp.float32) * (1.0 / math.sqrt(HID)))
    h_ref[...] = jax.nn.silu(jnp.dot(h, w2_ref[...],
                                     preferred_element_type=jnp.float32) * (1.0 / math.sqrt(HID)))


def _build_sc_scatter(E, N):
    n_tiles = 32
    ept = E // n_tiles
    C = 40
    n_chunks = ept // C
    NBUF = 5
    n_outer = n_chunks // NBUF
    rows = (N + 127) // 128 * 128 + 128
    rows_per_tile = rows // 16
    mesh = plsc.VectorSubcoreMesh(core_axis_name="c", subcore_axis_name="s")
    out_t = tuple(jax.ShapeDtypeStruct((2, rows, GW), jnp.float32) for _ in range(NPASS))

    n_outer2 = n_outer // 2

    @functools.partial(
        pl.kernel, out_type=out_t, mesh=mesh,
        compiler_params=pltpu.CompilerParams(needs_layout_passes=False),
        scratch_types=[pltpu.VMEM((C, GW), jnp.float32)] * 2
                      + [pltpu.VMEM((C, HID), jnp.float32)] * NBUF
                      + [pltpu.VMEM((C,), jnp.int32)] * (2 * NBUF)
                      + [pltpu.VMEM((C,), jnp.float32)] * (2 * NBUF)
                      + [pltpu.VMEM_SHARED((rows, GW), jnp.float32)]
                      + [pltpu.SemaphoreType.DMA] * (NBUF + 2))
    def sc_scatter(src_hbm, h_hbm, sh_hbm, zero_hbm,
                   o0, o1, o2, o3, o4, *rest):
        pays = rest[:2]
        hbufs = rest[2:2 + NBUF]
        idxs = [rest[2 + NBUF:2 + 2 * NBUF], rest[2 + 2 * NBUF:2 + 3 * NBUF]]
        sh0s = rest[2 + 3 * NBUF:2 + 4 * NBUF]
        sh1s = rest[2 + 4 * NBUF:2 + 5 * NBUF]
        accum = rest[2 + 5 * NBUF]
        gsems = rest[3 + 5 * NBUF:3 + 6 * NBUF]
        ssems = rest[3 + 6 * NBUF:]
        c = lax.axis_index("c")
        s = lax.axis_index("s")
        wid = c * 16 + s
        tile_base = wid * ept
        row0 = s * rows_per_tile

        for p, o_hbm in enumerate((o0, o1, o2, o3, o4)):
            pltpu.sync_copy(zero_hbm.at[pl.ds(row0, rows_per_tile)],
                            accum.at[pl.ds(row0, rows_per_tile)])
            plsc.subcore_barrier()

            def fetch(i, q, b):
                pltpu.async_copy(src_hbm.at[wid, i], idxs[q][b], gsems[b])
                pltpu.async_copy(h_hbm.at[pl.ds(tile_base + i * C, C)],
                                 hbufs[b], gsems[b])
                pltpu.async_copy(sh_hbm.at[2 * p, wid, i], sh0s[b], gsems[b])
                pltpu.async_copy(sh_hbm.at[2 * p + 1, wid, i], sh1s[b], gsems[b])

            for b in range(NBUF):
                fetch(b, 0, b)

            def compute(b, pb):
                @functools.partial(plsc.parallel_loop, 0, C, unroll=4)
                def edge(e):
                    e16 = jnp.full((16,), e, dtype=jnp.int32)
                    b0 = plsc.load_gather(sh0s[b], [e16])
                    b1 = plsc.load_gather(sh1s[b], [e16])
                    for j in range(HID // 16):
                        hv = hbufs[b][e, pl.ds(16 * j, 16)]
                        pays[pb][e, pl.ds(16 * j, 16)] = hv * b0
                        pays[pb][e, pl.ds(HID + 16 * j, 16)] = hv * b1

            def outer(j2, carry):
                for q in range(2):
                    for b in range(NBUF):
                        i = j2 * 2 * NBUF + q * NBUF + b
                        pb = (q * NBUF + b) % 2
                        pltpu.make_async_copy(src_hbm.at[0, 0], idxs[q][b],
                                              gsems[b]).wait()
                        pltpu.make_async_copy(h_hbm.at[pl.ds(0, C)], hbufs[b],
                                              gsems[b]).wait()
                        pltpu.make_async_copy(sh_hbm.at[0, 0, 0], sh0s[b],
                                              gsems[b]).wait()
                        pltpu.make_async_copy(sh_hbm.at[0, 0, 0], sh1s[b],
                                              gsems[b]).wait()
                        if q == 0 and b < 2:
                            @pl.when(j2 > 0)
                            def _():
                                pltpu.make_async_copy(
                                    pays[pb], accum.at[idxs[q][b]],
                                    ssems[pb]).wait()
                        else:
                            pltpu.make_async_copy(
                                pays[pb], accum.at[idxs[q][b]], ssems[pb]).wait()
                        compute(b, pb)
                        pltpu.async_copy(pays[pb], accum.at[idxs[q][b]],
                                         ssems[pb], add=True)
                        if q == 0:
                            fetch(i + NBUF, 1, b)
                        else:
                            @pl.when(j2 < n_outer2 - 1)
                            def _():
                                fetch(i + NBUF, 0, b)
                return carry

            lax.fori_loop(0, n_outer2, outer, 0)
            for pb in range(2):
                pltpu.make_async_copy(pays[pb], accum.at[idxs[1][0]],
                                      ssems[pb]).wait()
            plsc.subcore_barrier()
            pltpu.sync_copy(accum.at[pl.ds(row0, rows_per_tile)],
                            o_hbm.at[c, pl.ds(row0, rows_per_tile)])

    return sc_scatter


def _final_body(nf_ref, na_ref, *rest):
    t_refs = rest[:2 * NPASS]
    (wskip_ref, wlin_ref, wmlp3_ref, w20_ref, w21_ref, w22_ref,
     out9_ref, sc_ref) = rest[2 * NPASS:]
    nf = nf_ref[...]
    acc = jnp.zeros_like(nf)
    for v in range(NATTR):
        acc = acc + jnp.dot(nf, wskip_ref[:, v, :],
                            preferred_element_type=jnp.float32) * na_ref[:, v:v + 1]
    sc_ref[...] = acc * (1.0 / math.sqrt(MUL * NATTR))
    x = jnp.dot(nf, wlin_ref[...],
                preferred_element_type=jnp.float32) * (1.0 / math.sqrt(MUL))
    w2s = (w20_ref, w21_ref, w22_ref)
    scale = 1.0 / (math.sqrt(HID) * math.sqrt(MUL) * AVG_NUM_NEIGHBORS)
    for d in range(9):
        p, dl = divmod(d, 2)
        l = _L_OF_D[d]
        ta, tb = t_refs[2 * p], t_refs[2 * p + 1]
        td = (ta[0, :, HID * dl:HID * (dl + 1)]
              + tb[0, :, HID * dl:HID * (dl + 1)])
        m = jnp.dot(td, wmlp3_ref[:, l * MUL:(l + 1) * MUL],
                    preferred_element_type=jnp.float32)
        out9_ref[d] = jnp.dot(x * m, w2s[l][...],
                              preferred_element_type=jnp.float32) * scale


def kernel(node_feat, node_attr, edge_idx, edge_dist_embedding, edge_diff_embedding,
           W_skip, W_lin1, W_mlp0, W_mlp1, W_mlp2, W_mlp3, W2_0, W2_1, W2_2):
    N = node_feat.shape[0]
    E = edge_dist_embedding.shape[0]
    src = edge_idx[:, 0]
    rows = (N + 127) // 128 * 128 + 128

    EB = 8000
    h = pl.pallas_call(
        _mlp_body,
        grid=(E // EB,),
        in_specs=[
            pl.BlockSpec((EB, 8), lambda i: (i, 0)),
            pl.BlockSpec((8, HID), lambda i: (0, 0)),
            pl.BlockSpec((HID, HID), lambda i: (0, 0)),
            pl.BlockSpec((HID, HID), lambda i: (0, 0)),
        ],
        out_specs=pl.BlockSpec((EB, HID), lambda i: (i, 0)),
        out_shape=jax.ShapeDtypeStruct((E, HID), jnp.float32),
    )(edge_dist_embedding, W_mlp0, W_mlp1, W_mlp2)

    zeros = jnp.zeros((rows, GW), jnp.float32)
    shp = jnp.pad(edge_diff_embedding, ((0, 0), (0, 1)))
    sh10 = shp.T.reshape(10, 32, -1, 40)
    ts = _build_sc_scatter(E, N)(src.reshape(32, -1, 40), h, sh10, zeros)

    NB = 400
    t_specs = []
    t_args = []
    for t in ts:
        t_specs += [pl.BlockSpec((1, NB, GW), lambda i: (0, i, 0)),
                    pl.BlockSpec((1, NB, GW), lambda i: (1, i, 0))]
        t_args += [t, t]
    out9, sc = pl.pallas_call(
        _final_body,
        grid=(N // NB,),
        in_specs=[
            pl.BlockSpec((NB, MUL), lambda i: (i, 0)),
            pl.BlockSpec((NB, NATTR), lambda i: (i, 0)),
        ] + t_specs + [
            pl.BlockSpec((MUL, NATTR, MUL), lambda i: (0, 0, 0)),
            pl.BlockSpec((MUL, MUL), lambda i: (0, 0)),
            pl.BlockSpec((HID, 3 * MUL), lambda i: (0, 0)),
            pl.BlockSpec((MUL, MUL), lambda i: (0, 0)),
            pl.BlockSpec((MUL, MUL), lambda i: (0, 0)),
            pl.BlockSpec((MUL, MUL), lambda i: (0, 0)),
        ],
        out_specs=[
            pl.BlockSpec((9, NB, MUL), lambda i: (0, i, 0)),
            pl.BlockSpec((NB, MUL), lambda i: (i, 0)),
        ],
        out_shape=[
            jax.ShapeDtypeStruct((9, N, MUL), jnp.float32),
            jax.ShapeDtypeStruct((N, MUL), jnp.float32),
        ],
    )(node_feat, node_attr, *t_args,
      W_skip, W_lin1, W_mlp3, W2_0, W2_1, W2_2)

    return (jnp.transpose(out9, (1, 2, 0)), sc)
